# baseline (speedup 1.0000x reference)
import functools

import jax
import jax.numpy as jnp
from jax import lax
from jax.experimental import pallas as pl
from jax.experimental.pallas import tpu as pltpu
from jax.experimental.pallas import tpu_sc as plsc

_NCLS = 10
_B = 1024
_NW = 32
_BPW = _B // _NW
_RING = 8
_NREP = 2


def _make_kernel():
    mesh = plsc.VectorSubcoreMesh(core_axis_name="c", subcore_axis_name="s")

    @functools.partial(
        pl.kernel,
        mesh=mesh,
        out_type=jax.ShapeDtypeStruct((_B, 100, 784), jnp.float32),
        scratch_types=[
            pltpu.VMEM((_BPW,), jnp.int32),
            pltpu.VMEM_SHARED((_NREP * _NCLS, 100, 784), jnp.float32),
            pltpu.SemaphoreType.DMA((_RING,)),
        ],
    )
    def emb(y_hbm, a_hbm, out_hbm, y_v, table_s, sems):
        sid = lax.axis_index("s")
        wid = sid * 2 + lax.axis_index("c")
        base = wid * _BPW

        pltpu.sync_copy(y_hbm.at[pl.ds(base, _BPW)], y_v)

        @pl.when(sid < _NCLS)
        def _():
            pltpu.sync_copy(a_hbm.at[pl.ds(sid, 1)], table_s.at[pl.ds(sid, 1)])

        @pl.when((sid >= _NCLS) | (sid < _NREP * _NCLS - 16))
        def _():
            blk = jnp.where(sid >= _NCLS, sid, sid + 16)
            pltpu.sync_copy(a_hbm.at[pl.ds(blk - _NCLS, 1)],
                            table_s.at[pl.ds(blk, 1)])

        plsc.subcore_barrier()

        rep = lax.rem(sid, _NREP) * _NCLS

        def wrblock(row, i, slot):
            return pltpu.make_async_copy(table_s.at[pl.ds(row, 1)],
                                         out_hbm.at[pl.ds(i, 1)],
                                         sems.at[slot])

        def block(g, _):
            y16 = y_v[pl.ds(g * 16, 16)]
            for j in range(16):
                t = g * 16 + j
                row = rep + y16[j]
                slot = lax.rem(t, _RING)

                @pl.when(t >= _RING)
                def _():
                    wrblock(row, base + t, slot).wait()

                wrblock(row, base + t, slot).start()
            return ()

        lax.fori_loop(0, _BPW // 16, block, ())

        for s in range(_RING):
            wrblock(0, 0, s).wait()

    return emb


_emb = _make_kernel()


def kernel(y, A):
    a3 = A.reshape(_NCLS, 100, 784)
    return _emb(y.astype(jnp.int32), a3)


# speedup vs baseline: 1.0920x; 1.0920x over previous
import functools

import jax
import jax.numpy as jnp
from jax import lax
from jax.experimental import pallas as pl
from jax.experimental.pallas import tpu as pltpu
from jax.experimental.pallas import tpu_sc as plsc

_NCLS = 10
_B = 1024
_NW = 32
_RING = 8


def _make_kernel():
    mesh = plsc.VectorSubcoreMesh(core_axis_name="c", subcore_axis_name="s")

    @functools.partial(
        pl.kernel,
        mesh=mesh,
        out_type=jax.ShapeDtypeStruct((_B, 100, 784), jnp.float32),
        scratch_types=[
            pltpu.VMEM((_B,), jnp.int32),
            pltpu.VMEM((1, 100, 784), jnp.float32),
            pltpu.SemaphoreType.DMA((_RING,)),
        ],
    )
    def emb(y_hbm, a_hbm, out_hbm, y_v, buf, sems):
        wid = lax.axis_index("s") * 2 + lax.axis_index("c")

        pltpu.sync_copy(y_hbm, y_v)

        cnt = [jnp.int32(0)] * _NCLS

        def count_block(g, carry):
            y16 = y_v[pl.ds(g * 16, 16)]
            out = list(carry)
            for j in range(16):
                c = y16[j]
                for k in range(_NCLS):
                    out[k] = out[k] + jnp.where(c == k, 1, 0)
            return tuple(out)

        cnt = list(lax.fori_loop(0, _B // 16, count_block, tuple(cnt)))

        k = [jnp.where(cnt[c] > 0, 1, 0) for c in range(_NCLS)]
        used = k[0]
        for c in range(1, _NCLS):
            used = used + k[c]
        for _ in range(_NW - 1):
            have = used < _NW
            best_c = jnp.int32(0)
            best_p = jnp.int32(-1)
            for c in range(_NCLS):
                p = cnt[c] * 1024 // (k[c] + 1)
                take = p > best_p
                best_p = jnp.where(take, p, best_p)
                best_c = jnp.where(take, jnp.int32(c), best_c)
            for c in range(_NCLS):
                k[c] = k[c] + jnp.where(have & (best_c == c), 1, 0)
            used = used + jnp.where(have, 1, 0)

        csum = jnp.int32(0)
        mycls = jnp.int32(0)
        prevcs = jnp.int32(0)
        for c in range(_NCLS):
            nxt = csum + k[c]
            inrange = (wid >= csum) & (wid < nxt)
            mycls = jnp.where(inrange, jnp.int32(c), mycls)
            prevcs = jnp.where(inrange, csum, prevcs)
            csum = nxt
        myrank = wid - prevcs
        ntiles = jnp.int32(0)
        for c in range(_NCLS):
            ntiles = ntiles + jnp.where(mycls == c, k[c], 0)
        active = wid < csum

        @pl.when(active)
        def _():
            pltpu.sync_copy(a_hbm.at[pl.ds(mycls, 1)], buf)

        def wrblock(i, slot):
            return pltpu.make_async_copy(buf, out_hbm.at[pl.ds(i, 1)],
                                         sems.at[slot])

        def block(g, carry):
            n, cn = carry
            y16 = y_v[pl.ds(g * 16, 16)]
            for j in range(16):
                hit = active & (y16[j] == mycls)
                mine = hit & (lax.rem(cn, ntiles + (1 - active)) == myrank)
                slot = lax.rem(n, _RING)
                i = g * 16 + j

                @pl.when(mine & (n >= _RING))
                def _():
                    wrblock(i, slot).wait()

                @pl.when(mine)
                def _():
                    wrblock(i, slot).start()

                n = n + jnp.where(mine, 1, 0)
                cn = cn + jnp.where(hit, 1, 0)
            return n, cn

        n, _ = lax.fori_loop(0, _B // 16, block, (jnp.int32(0), jnp.int32(0)))

        for s in range(_RING):
            @pl.when(n > s)
            def _():
                wrblock(0, s).wait()

    return emb


_emb = _make_kernel()


def kernel(y, A):
    a3 = A.reshape(_NCLS, 100, 784)
    return _emb(y.astype(jnp.int32), a3)


# speedup vs baseline: 1.2087x; 1.1069x over previous
import jax
import jax.numpy as jnp
from jax import lax
from jax.experimental import pallas as pl
from jax.experimental.pallas import tpu as pltpu

_NCLS = 10
_B = 1024
_K = 16


def _body(y_sp, a_ref, o_ref, sems):
    def start(i):
        pltpu.make_async_copy(a_ref.at[y_sp[i]], o_ref.at[i],
                              sems.at[i % _K]).start()

    def wait(i):
        pltpu.make_async_copy(a_ref.at[0], o_ref.at[i],
                              sems.at[i % _K]).wait()

    for i in range(_K):
        start(i)

    def loop(i, _):
        wait(i - _K)
        start(i)
        return ()

    lax.fori_loop(_K, _B, loop, ())

    for i in range(_B - _K, _B):
        wait(i)


def kernel(y, A):
    a3 = A.reshape(_NCLS, 100, 784)
    out = pl.pallas_call(
        _body,
        grid_spec=pltpu.PrefetchScalarGridSpec(
            num_scalar_prefetch=1,
            grid=(1,),
            in_specs=[pl.BlockSpec((_NCLS, 100, 784), lambda i, y_sp: (0, 0, 0))],
            out_specs=pl.BlockSpec(memory_space=pl.ANY),
            scratch_shapes=[pltpu.SemaphoreType.DMA((_K,))],
        ),
        out_shape=jax.ShapeDtypeStruct((_B, 100, 784), jnp.float32),
        compiler_params=pltpu.CompilerParams(dimension_semantics=("arbitrary",)),
    )(y.astype(jnp.int32), a3)
    return out
